# baseline (speedup 1.0000x reference)
import jax
import jax.numpy as jnp
from jax.experimental import pallas as pl
from jax.experimental.pallas import tpu as pltpu

D_IN = 200000
H0 = 512
H1 = 512
N_ACT = 200002
BATCH = 8

NSPLIT = 4
K_BLK = 4000
K_SUB = K_BLK // NSPLIT
N_BLK = 8192
H_SUB = H1 // NSPLIT
P1 = D_IN // K_BLK
P2 = (N_ACT + N_BLK - 1) // N_BLK


def _fused_kernel(x_ref, w0a_ref, w0b_ref, w0c_ref, w0d_ref,
                  b0_ref, w1_ref, b1_ref,
                  w2a_ref, w2b_ref, w2c_ref, w2d_ref, b2_ref,
                  o_ref, acc_ref, h_ref):
    i = pl.program_id(0)

    @pl.when(i == 0)
    def _init():
        acc_ref[...] = jnp.zeros_like(acc_ref)

    @pl.when(i < P1)
    def _layer1():
        part = jnp.zeros((BATCH, H0), jnp.float32)
        for s, w_ref in enumerate((w0a_ref, w0b_ref, w0c_ref, w0d_ref)):
            xs = x_ref[:, 0, s, :]
            part += jnp.dot(xs, w_ref[0, 0],
                            preferred_element_type=jnp.float32)
        acc_ref[...] += part

    @pl.when(i == P1 - 1)
    def _layer2():
        h0 = jnp.maximum(acc_ref[...] + b0_ref[...], 0.0)
        h1 = jnp.dot(h0, w1_ref[...], preferred_element_type=jnp.float32)
        h_ref[...] = jnp.maximum(h1 + b1_ref[...], 0.0)

    @pl.when(i >= P1)
    def _layer3():
        h = h_ref[...]
        out = b2_ref[...].astype(jnp.float32)
        for s, w_ref in enumerate((w2a_ref, w2b_ref, w2c_ref, w2d_ref)):
            out += jnp.dot(h[:, s * H_SUB:(s + 1) * H_SUB], w_ref[0],
                           preferred_element_type=jnp.float32)
        o_ref[...] = out


def kernel(state, W0, b0, W1, b1, W2, b2):
    xr = state.reshape(BATCH, P1, NSPLIT, K_SUB)
    w0r = W0.reshape(P1, NSPLIT, K_SUB, H0)
    w2r = W2.reshape(NSPLIT, H_SUB, N_ACT)
    b0r = b0.reshape(1, H0)
    b1r = b1.reshape(1, H1)
    b2r = b2.reshape(1, N_ACT)

    def w0_spec(s):
        return pl.BlockSpec((1, 1, K_SUB, H0),
                            lambda i, s=s: (jnp.minimum(i, P1 - 1), s, 0, 0))

    def w2_spec(s):
        return pl.BlockSpec((1, H_SUB, N_BLK),
                            lambda i, s=s: (s, 0, jnp.maximum(i - P1, 0)))

    logits = pl.pallas_call(
        _fused_kernel,
        grid=(P1 + P2,),
        in_specs=[
            pl.BlockSpec((BATCH, 1, NSPLIT, K_SUB),
                         lambda i: (0, jnp.minimum(i, P1 - 1), 0, 0)),
            w0_spec(0), w0_spec(1), w0_spec(2), w0_spec(3),
            pl.BlockSpec((1, H0), lambda i: (0, 0)),
            pl.BlockSpec((H0, H1), lambda i: (0, 0)),
            pl.BlockSpec((1, H1), lambda i: (0, 0)),
            w2_spec(0), w2_spec(1), w2_spec(2), w2_spec(3),
            pl.BlockSpec((1, N_BLK), lambda i: (0, jnp.maximum(i - P1, 0))),
        ],
        out_specs=pl.BlockSpec((BATCH, N_BLK),
                               lambda i: (0, jnp.maximum(i - P1, 0))),
        out_shape=jax.ShapeDtypeStruct((BATCH, N_ACT), jnp.float32),
        scratch_shapes=[
            pltpu.VMEM((BATCH, H0), jnp.float32),
            pltpu.VMEM((BATCH, H1), jnp.float32),
        ],
        compiler_params=pltpu.CompilerParams(
            dimension_semantics=("arbitrary",)),
    )(xr, w0r, w0r, w0r, w0r, b0r, W1, b1r, w2r, w2r, w2r, w2r, b2r)
    return logits


# speedup vs baseline: 1.0053x; 1.0053x over previous
import jax
import jax.numpy as jnp
from jax.experimental import pallas as pl
from jax.experimental.pallas import tpu as pltpu

D_IN = 200000
H0 = 512
H1 = 512
N_ACT = 200002
BATCH = 8

NSPLIT = 4
K_BLK = 4000
K_SUB = K_BLK // NSPLIT
N_BLK = 4096
H_SUB = H1 // NSPLIT
P1 = D_IN // K_BLK
P2 = (N_ACT + N_BLK - 1) // N_BLK


def _fused_kernel(x_ref, w0a_ref, w0b_ref, w0c_ref, w0d_ref,
                  b0_ref, w1_ref, b1_ref,
                  w2a_ref, w2b_ref, w2c_ref, w2d_ref, b2_ref,
                  o_ref, acc_ref, h_ref):
    i = pl.program_id(0)

    @pl.when(i == 0)
    def _init():
        acc_ref[...] = jnp.zeros_like(acc_ref)

    @pl.when(i < P1)
    def _layer1():
        part = jnp.zeros((BATCH, H0), jnp.float32)
        for s, w_ref in enumerate((w0a_ref, w0b_ref, w0c_ref, w0d_ref)):
            xs = x_ref[:, 0, s, :]
            part += jnp.dot(xs, w_ref[0, 0],
                            preferred_element_type=jnp.float32)
        acc_ref[...] += part

    @pl.when(i == P1 - 1)
    def _layer2():
        h0 = jnp.maximum(acc_ref[...] + b0_ref[...], 0.0)
        h1 = jnp.dot(h0, w1_ref[...], preferred_element_type=jnp.float32)
        h_ref[...] = jnp.maximum(h1 + b1_ref[...], 0.0)

    @pl.when(i >= P1)
    def _layer3():
        h = h_ref[...]
        out = b2_ref[...].astype(jnp.float32)
        for s, w_ref in enumerate((w2a_ref, w2b_ref, w2c_ref, w2d_ref)):
            out += jnp.dot(h[:, s * H_SUB:(s + 1) * H_SUB], w_ref[0],
                           preferred_element_type=jnp.float32)
        o_ref[...] = out


def kernel(state, W0, b0, W1, b1, W2, b2):
    xr = state.reshape(BATCH, P1, NSPLIT, K_SUB)
    w0r = W0.reshape(P1, NSPLIT, K_SUB, H0)
    w2r = W2.reshape(NSPLIT, H_SUB, N_ACT)
    b0r = b0.reshape(1, H0)
    b1r = b1.reshape(1, H1)
    b2r = b2.reshape(1, N_ACT)

    def w0_spec(s):
        return pl.BlockSpec((1, 1, K_SUB, H0),
                            lambda i, s=s: (jnp.minimum(i, P1 - 1), s, 0, 0))

    def w2_spec(s):
        return pl.BlockSpec((1, H_SUB, N_BLK),
                            lambda i, s=s: (s, 0, jnp.maximum(i - P1, 0)))

    logits = pl.pallas_call(
        _fused_kernel,
        grid=(P1 + P2,),
        in_specs=[
            pl.BlockSpec((BATCH, 1, NSPLIT, K_SUB),
                         lambda i: (0, jnp.minimum(i, P1 - 1), 0, 0)),
            w0_spec(0), w0_spec(1), w0_spec(2), w0_spec(3),
            pl.BlockSpec((1, H0), lambda i: (0, 0)),
            pl.BlockSpec((H0, H1), lambda i: (0, 0)),
            pl.BlockSpec((1, H1), lambda i: (0, 0)),
            w2_spec(0), w2_spec(1), w2_spec(2), w2_spec(3),
            pl.BlockSpec((1, N_BLK), lambda i: (0, jnp.maximum(i - P1, 0))),
        ],
        out_specs=pl.BlockSpec((BATCH, N_BLK),
                               lambda i: (0, jnp.maximum(i - P1, 0))),
        out_shape=jax.ShapeDtypeStruct((BATCH, N_ACT), jnp.float32),
        scratch_shapes=[
            pltpu.VMEM((BATCH, H0), jnp.float32),
            pltpu.VMEM((BATCH, H1), jnp.float32),
        ],
        compiler_params=pltpu.CompilerParams(
            dimension_semantics=("arbitrary",)),
    )(xr, w0r, w0r, w0r, w0r, b0r, W1, b1r, w2r, w2r, w2r, w2r, b2r)
    return logits
